# baseline (speedup 1.0000x reference)
import functools

import jax
import jax.numpy as jnp
from jax import lax
from jax.experimental import pallas as pl
from jax.experimental.pallas import tpu as pltpu
from jax.experimental.pallas import tpu_sc as plsc

N = 10000
E = 320000
D = 128
K = 16

NC = 2
NS = 16
NW = NC * NS
CHUNK = 40
EDGES_PER_TILE = E // NW
NCHUNKS = EDGES_PER_TILE // CHUNK
SUPER = 25
NSUPER = NCHUNKS // SUPER
NBUF = 4
NPAD = 10240
ROWS_PER_TILE = NPAD // NS

_SELU_SCALE = 1.0507009873554805
_SELU_ALPHA = 1.6732632423543772


def _selu(x):
    safe = jnp.minimum(x, 0.0)
    return _SELU_SCALE * jnp.where(x > 0, x, _SELU_ALPHA * (jnp.exp(safe) - 1.0))


def _sc_segment_sum(y, edges_r, zeros_nd):
    mesh = plsc.VectorSubcoreMesh(core_axis_name="c", subcore_axis_name="s")

    @functools.partial(
        pl.kernel,
        mesh=mesh,
        out_type=jax.ShapeDtypeStruct((NC, NPAD, D), jnp.float32),
        scratch_types=[
            pltpu.VMEM((SUPER, CHUNK), jnp.int32),
            pltpu.VMEM((SUPER, CHUNK), jnp.int32),
            pltpu.VMEM((SUPER, CHUNK), jnp.int32),
            pltpu.VMEM((SUPER, CHUNK), jnp.int32),
        ] + [pltpu.VMEM((CHUNK, D), jnp.float32) for _ in range(NBUF)] + [
            pltpu.VMEM_SHARED((NPAD, D), jnp.float32),
        ] + [pltpu.SemaphoreType.DMA for _ in range(NBUF)] + [
            pltpu.SemaphoreType.DMA,
            pltpu.SemaphoreType.DMA,
        ],
    )
    def k(y_hbm, e_hbm, z_hbm, out_hbm, src_a, src_b, dst_a, dst_b,
          rows0, rows1, rows2, rows3, acc_sh, gsem0, gsem1, gsem2, gsem3,
          isem_a, isem_b):
        rows = (rows0, rows1, rows2, rows3)
        gsem = (gsem0, gsem1, gsem2, gsem3)
        cid = lax.axis_index("c")
        sid = lax.axis_index("s")
        wid = cid * NS + sid
        r0 = sid * ROWS_PER_TILE
        pltpu.sync_copy(z_hbm.at[pl.ds(r0, ROWS_PER_TILE)],
                        acc_sh.at[pl.ds(r0, ROWS_PER_TILE)])

        def idx_bufs(s):
            return (src_a, dst_a, isem_a) if s % 2 == 0 else \
                   (src_b, dst_b, isem_b)

        s0, d0, i0 = idx_bufs(0)
        pltpu.async_copy(e_hbm.at[0, wid, 0], s0, i0)
        pltpu.async_copy(e_hbm.at[1, wid, 0], d0, i0)
        plsc.subcore_barrier()

        for s in range(NSUPER):
            src_s, dst_s, is_s = idx_bufs(s)
            pltpu.make_async_copy(e_hbm.at[0, wid, s], src_s, is_s).wait()
            pltpu.make_async_copy(e_hbm.at[1, wid, s], dst_s, is_s).wait()
            if s + 1 < NSUPER:
                sn, dn, i_n = idx_bufs(s + 1)
                pltpu.async_copy(e_hbm.at[0, wid, s + 1], sn, i_n)
                pltpu.async_copy(e_hbm.at[1, wid, s + 1], dn, i_n)

            for c in range(NBUF - 1):
                pltpu.async_copy(y_hbm.at[src_s.at[c]], rows[c], gsem[c])

            @pl.loop(0, SUPER - 1, step=NBUF)
            def _(j):
                for b in range(NBUF):
                    pltpu.make_async_copy(y_hbm.at[src_s.at[j + b]], rows[b],
                                          gsem[b]).wait()
                    pltpu.sync_copy(rows[b], acc_sh.at[dst_s.at[j + b]],
                                    add=True)

                    @pl.when(j + b + NBUF - 1 < SUPER)
                    def _():
                        nb = (b + NBUF - 1) % NBUF
                        pltpu.async_copy(y_hbm.at[src_s.at[j + b + NBUF - 1]],
                                         rows[nb], gsem[nb])

            pltpu.make_async_copy(y_hbm.at[src_s.at[SUPER - 1]],
                                  rows[(SUPER - 1) % NBUF],
                                  gsem[(SUPER - 1) % NBUF]).wait()
            pltpu.sync_copy(rows[(SUPER - 1) % NBUF],
                            acc_sh.at[dst_s.at[SUPER - 1]], add=True)

        plsc.subcore_barrier()
        pltpu.sync_copy(acc_sh.at[pl.ds(r0, ROWS_PER_TILE)],
                        out_hbm.at[cid, pl.ds(r0, ROWS_PER_TILE)])

    return k(y, edges_r, zeros_nd)


_RB2 = 1000
_GB = N // _RB2


def _epilogue_body(x_ref, p0_ref, p1_ref, w_ref, sw_ref, b_ref, wt_ref,
                   bt_ref, a_ref, pooled_ref, s_ref, sz_ref):
    i = pl.program_id(0)

    @pl.when(i == 0)
    def _():
        s_ref[...] = jnp.zeros_like(s_ref)
        sz_ref[...] = jnp.zeros_like(sz_ref)

    y = jnp.dot(x_ref[...], w_ref[...], preferred_element_type=jnp.float32)
    agg = jnp.dot(p0_ref[0] + p1_ref[0], w_ref[...],
                  preferred_element_type=jnp.float32)
    h = _selu(y * sw_ref[...] + agg + b_ref[...])
    logits = jnp.dot(h, wt_ref[...],
                     preferred_element_type=jnp.float32) + bt_ref[...]
    m = jnp.max(logits, axis=-1, keepdims=True)
    e = jnp.exp(logits - m)
    a = e / jnp.sum(e, axis=-1, keepdims=True)
    a_ref[...] = a
    s_ref[...] += lax.dot_general(a, h, (((0,), (0,)), ((), ())),
                                  preferred_element_type=jnp.float32)
    ones = jnp.ones((_RB2, D), jnp.float32)
    sz_ref[...] += lax.dot_general(a, ones, (((0,), (0,)), ((), ())),
                                   preferred_element_type=jnp.float32)

    @pl.when(i == _GB - 1)
    def _():
        pooled_ref[...] = _selu(s_ref[...] / sz_ref[...])


def _epilogue(x, partials, W, skip_w, b, Wt, bt):
    a, pooled = pl.pallas_call(
        _epilogue_body,
        grid=(_GB,),
        in_specs=[
            pl.BlockSpec((_RB2, D), lambda i: (i, 0)),
            pl.BlockSpec((1, _RB2, D), lambda i: (0, i, 0)),
            pl.BlockSpec((1, _RB2, D), lambda i: (1, i, 0)),
            pl.BlockSpec((D, D), lambda i: (0, 0)),
            pl.BlockSpec((1, D), lambda i: (0, 0)),
            pl.BlockSpec((1, D), lambda i: (0, 0)),
            pl.BlockSpec((D, K), lambda i: (0, 0)),
            pl.BlockSpec((1, K), lambda i: (0, 0)),
        ],
        out_specs=[
            pl.BlockSpec((_RB2, K), lambda i: (i, 0)),
            pl.BlockSpec((K, D), lambda i: (0, 0)),
        ],
        out_shape=[
            jax.ShapeDtypeStruct((N, K), jnp.float32),
            jax.ShapeDtypeStruct((K, D), jnp.float32),
        ],
        scratch_shapes=[
            pltpu.VMEM((K, D), jnp.float32),
            pltpu.VMEM((K, D), jnp.float32),
        ],
    )(x, partials, partials, W, skip_w.reshape(1, D), b.reshape(1, D), Wt,
      bt.reshape(1, K))
    return pooled, a


def kernel(features, edges, adjacency, W, b, skip_w, Wt, bt):
    edges_r = edges.reshape(2, NW, NSUPER, SUPER, CHUNK)
    zeros_nd = jnp.zeros((NPAD, D), jnp.float32)
    partials = _sc_segment_sum(features, edges_r, zeros_nd)
    pooled, a = _epilogue(features, partials, W, skip_w, b, Wt, bt)
    return pooled, a


# speedup vs baseline: 1.1055x; 1.1055x over previous
import functools

import jax
import jax.numpy as jnp
from jax import lax
from jax.experimental import pallas as pl
from jax.experimental.pallas import tpu as pltpu
from jax.experimental.pallas import tpu_sc as plsc

N = 10000
E = 320000
D = 128
K = 16

NC = 2
NS = 16
NW = NC * NS
CHUNK = 80
EDGES_PER_TILE = E // NW
NCHUNKS = EDGES_PER_TILE // CHUNK
SUPER = 25
NSUPER = NCHUNKS // SUPER
NPAD = 10240
ROWS_PER_TILE = NPAD // NS

_SELU_SCALE = 1.0507009873554805
_SELU_ALPHA = 1.6732632423543772


def _selu(x):
    safe = jnp.minimum(x, 0.0)
    return _SELU_SCALE * jnp.where(x > 0, x, _SELU_ALPHA * (jnp.exp(safe) - 1.0))


def _sc_segment_sum(y, edges_r):
    mesh = plsc.VectorSubcoreMesh(core_axis_name="c", subcore_axis_name="s")

    @functools.partial(
        pl.kernel,
        mesh=mesh,
        out_type=jax.ShapeDtypeStruct((NC, NPAD, D), jnp.float32),
        scratch_types=[
            pltpu.VMEM((SUPER, CHUNK), jnp.int32),
            pltpu.VMEM((SUPER, CHUNK), jnp.int32),
            pltpu.VMEM((SUPER, CHUNK), jnp.int32),
            pltpu.VMEM((SUPER, CHUNK), jnp.int32),
            pltpu.VMEM((CHUNK, D), jnp.float32),
            pltpu.VMEM((CHUNK, D), jnp.float32),
            pltpu.VMEM_SHARED((NPAD, D), jnp.float32),
            pltpu.SemaphoreType.DMA,
            pltpu.SemaphoreType.DMA,
            pltpu.SemaphoreType.DMA,
            pltpu.SemaphoreType.DMA,
        ],
    )
    def k(y_hbm, e_hbm, out_hbm, src_a, src_b, dst_a, dst_b,
          rows0, rows1, acc_sh, gsem0, gsem1, isem_a, isem_b):
        cid = lax.axis_index("c")
        sid = lax.axis_index("s")
        wid = cid * NS + sid
        r0 = sid * ROWS_PER_TILE
        @pl.loop(0, CHUNK)
        def _(r):
            @pl.loop(0, D // 16)
            def _(c):
                rows0[r, pl.ds(c * 16, 16)] = jnp.zeros((16,), jnp.float32)

        @pl.loop(0, ROWS_PER_TILE // CHUNK)
        def _(i):
            pltpu.sync_copy(rows0, acc_sh.at[pl.ds(r0 + i * CHUNK, CHUNK)])

        def idx_bufs(s):
            return (src_a, dst_a, isem_a) if s % 2 == 0 else \
                   (src_b, dst_b, isem_b)

        s0, d0, i0 = idx_bufs(0)
        pltpu.async_copy(e_hbm.at[0, wid, 0], s0, i0)
        pltpu.async_copy(e_hbm.at[1, wid, 0], d0, i0)
        plsc.subcore_barrier()

        for s in range(NSUPER):
            src_s, dst_s, is_s = idx_bufs(s)
            pltpu.make_async_copy(e_hbm.at[0, wid, s], src_s, is_s).wait()
            pltpu.make_async_copy(e_hbm.at[1, wid, s], dst_s, is_s).wait()
            if s + 1 < NSUPER:
                sn, dn, i_n = idx_bufs(s + 1)
                pltpu.async_copy(e_hbm.at[0, wid, s + 1], sn, i_n)
                pltpu.async_copy(e_hbm.at[1, wid, s + 1], dn, i_n)

            pltpu.async_copy(y_hbm.at[src_s.at[0]], rows0, gsem0)

            @pl.loop(0, SUPER - 1, step=2)
            def _(j):
                pltpu.async_copy(y_hbm.at[src_s.at[j + 1]], rows1, gsem1)
                pltpu.make_async_copy(y_hbm.at[src_s.at[j]], rows0,
                                      gsem0).wait()
                pltpu.sync_copy(rows0, acc_sh.at[dst_s.at[j]], add=True)
                pltpu.async_copy(y_hbm.at[src_s.at[j + 2]], rows0, gsem0)
                pltpu.make_async_copy(y_hbm.at[src_s.at[j + 1]], rows1,
                                      gsem1).wait()
                pltpu.sync_copy(rows1, acc_sh.at[dst_s.at[j + 1]], add=True)

            pltpu.make_async_copy(y_hbm.at[src_s.at[SUPER - 1]], rows0,
                                  gsem0).wait()
            pltpu.sync_copy(rows0, acc_sh.at[dst_s.at[SUPER - 1]], add=True)

        plsc.subcore_barrier()
        pltpu.sync_copy(acc_sh.at[pl.ds(r0, ROWS_PER_TILE)],
                        out_hbm.at[cid, pl.ds(r0, ROWS_PER_TILE)])

    return k(y, edges_r)


_RB2 = 1000
_GB = N // _RB2


def _epilogue_body(x_ref, p0_ref, p1_ref, w_ref, sw_ref, b_ref, wt_ref,
                   bt_ref, a_ref, pooled_ref, s_ref, sz_ref):
    i = pl.program_id(0)

    @pl.when(i == 0)
    def _():
        s_ref[...] = jnp.zeros_like(s_ref)
        sz_ref[...] = jnp.zeros_like(sz_ref)

    y = jnp.dot(x_ref[...], w_ref[...], preferred_element_type=jnp.float32)
    agg = jnp.dot(p0_ref[0] + p1_ref[0], w_ref[...],
                  preferred_element_type=jnp.float32)
    h = _selu(y * sw_ref[...] + agg + b_ref[...])
    logits = jnp.dot(h, wt_ref[...],
                     preferred_element_type=jnp.float32) + bt_ref[...]
    m = jnp.max(logits, axis=-1, keepdims=True)
    e = jnp.exp(logits - m)
    a = e / jnp.sum(e, axis=-1, keepdims=True)
    a_ref[...] = a
    s_ref[...] += lax.dot_general(a, h, (((0,), (0,)), ((), ())),
                                  preferred_element_type=jnp.float32)
    ones = jnp.ones((_RB2, D), jnp.float32)
    sz_ref[...] += lax.dot_general(a, ones, (((0,), (0,)), ((), ())),
                                   preferred_element_type=jnp.float32)

    @pl.when(i == _GB - 1)
    def _():
        pooled_ref[...] = _selu(s_ref[...] / sz_ref[...])


def _epilogue(x, partials, W, skip_w, b, Wt, bt):
    a, pooled = pl.pallas_call(
        _epilogue_body,
        grid=(_GB,),
        in_specs=[
            pl.BlockSpec((_RB2, D), lambda i: (i, 0)),
            pl.BlockSpec((1, _RB2, D), lambda i: (0, i, 0)),
            pl.BlockSpec((1, _RB2, D), lambda i: (1, i, 0)),
            pl.BlockSpec((D, D), lambda i: (0, 0)),
            pl.BlockSpec((1, D), lambda i: (0, 0)),
            pl.BlockSpec((1, D), lambda i: (0, 0)),
            pl.BlockSpec((D, K), lambda i: (0, 0)),
            pl.BlockSpec((1, K), lambda i: (0, 0)),
        ],
        out_specs=[
            pl.BlockSpec((_RB2, K), lambda i: (i, 0)),
            pl.BlockSpec((K, D), lambda i: (0, 0)),
        ],
        out_shape=[
            jax.ShapeDtypeStruct((N, K), jnp.float32),
            jax.ShapeDtypeStruct((K, D), jnp.float32),
        ],
        scratch_shapes=[
            pltpu.VMEM((K, D), jnp.float32),
            pltpu.VMEM((K, D), jnp.float32),
        ],
    )(x, partials, partials, W, skip_w.reshape(1, D), b.reshape(1, D), Wt,
      bt.reshape(1, K))
    return pooled, a


def kernel(features, edges, adjacency, W, b, skip_w, Wt, bt):
    edges_r = edges.reshape(2, NW, NSUPER, SUPER, CHUNK)
    partials = _sc_segment_sum(features, edges_r)
    pooled, a = _epilogue(features, partials, W, skip_w, b, Wt, bt)
    return pooled, a


# speedup vs baseline: 1.1286x; 1.0209x over previous
import functools

import jax
import jax.numpy as jnp
from jax import lax
from jax.experimental import pallas as pl
from jax.experimental.pallas import tpu as pltpu
from jax.experimental.pallas import tpu_sc as plsc

N = 10000
E = 320000
D = 128
K = 16

NC = 2
NS = 16
NW = NC * NS
CHUNK = 80
EDGES_PER_TILE = E // NW
NCHUNKS = EDGES_PER_TILE // CHUNK
SUPER = 25
NSUPER = NCHUNKS // SUPER
NPAD = 10240
ROWS_PER_TILE = NPAD // NS

_SELU_SCALE = 1.0507009873554805
_SELU_ALPHA = 1.6732632423543772


def _selu(x):
    safe = jnp.minimum(x, 0.0)
    return _SELU_SCALE * jnp.where(x > 0, x, _SELU_ALPHA * (jnp.exp(safe) - 1.0))


def _sc_segment_sum(y, edges_r):
    mesh = plsc.VectorSubcoreMesh(core_axis_name="c", subcore_axis_name="s")

    @functools.partial(
        pl.kernel,
        mesh=mesh,
        out_type=jax.ShapeDtypeStruct((NC, NPAD, D), jnp.float32),
        scratch_types=[
            pltpu.VMEM((SUPER, CHUNK), jnp.int32),
            pltpu.VMEM((SUPER, CHUNK), jnp.int32),
            pltpu.VMEM((SUPER, CHUNK), jnp.int32),
            pltpu.VMEM((SUPER, CHUNK), jnp.int32),
            pltpu.VMEM((CHUNK, D), jnp.float32),
            pltpu.VMEM((CHUNK, D), jnp.float32),
            pltpu.VMEM_SHARED((NPAD, D), jnp.float32),
            pltpu.SemaphoreType.DMA,
            pltpu.SemaphoreType.DMA,
            pltpu.SemaphoreType.DMA,
            pltpu.SemaphoreType.DMA,
        ],
    )
    def k(y_hbm, e_hbm, out_hbm, src_a, src_b, dst_a, dst_b,
          rows0, rows1, acc_sh, gsem0, gsem1, isem_a, isem_b):
        cid = lax.axis_index("c")
        sid = lax.axis_index("s")
        wid = cid * NS + sid
        r0 = sid * ROWS_PER_TILE
        @pl.loop(0, CHUNK)
        def _(r):
            @pl.loop(0, D // 16)
            def _(c):
                rows0[r, pl.ds(c * 16, 16)] = jnp.zeros((16,), jnp.float32)

        @pl.loop(0, ROWS_PER_TILE // CHUNK)
        def _(i):
            pltpu.sync_copy(rows0, acc_sh.at[pl.ds(r0 + i * CHUNK, CHUNK)])

        def idx_bufs(s):
            return (src_a, dst_a, isem_a) if s % 2 == 0 else \
                   (src_b, dst_b, isem_b)

        s0, d0, i0 = idx_bufs(0)
        pltpu.async_copy(e_hbm.at[0, wid, 0], s0, i0)
        pltpu.async_copy(e_hbm.at[1, wid, 0], d0, i0)
        plsc.subcore_barrier()

        for s in range(NSUPER):
            src_s, dst_s, is_s = idx_bufs(s)
            pltpu.make_async_copy(e_hbm.at[0, wid, s], src_s, is_s).wait()
            pltpu.make_async_copy(e_hbm.at[1, wid, s], dst_s, is_s).wait()
            if s + 1 < NSUPER:
                sn, dn, i_n = idx_bufs(s + 1)
                pltpu.async_copy(e_hbm.at[0, wid, s + 1], sn, i_n)
                pltpu.async_copy(e_hbm.at[1, wid, s + 1], dn, i_n)

            pltpu.async_copy(y_hbm.at[src_s.at[0]], rows0, gsem0)

            @pl.loop(0, SUPER - 1, step=2)
            def _(j):
                pltpu.async_copy(y_hbm.at[src_s.at[j + 1]], rows1, gsem1)
                pltpu.make_async_copy(y_hbm.at[src_s.at[j]], rows0,
                                      gsem0).wait()
                pltpu.sync_copy(rows0, acc_sh.at[dst_s.at[j]], add=True)
                pltpu.async_copy(y_hbm.at[src_s.at[j + 2]], rows0, gsem0)
                pltpu.make_async_copy(y_hbm.at[src_s.at[j + 1]], rows1,
                                      gsem1).wait()
                pltpu.sync_copy(rows1, acc_sh.at[dst_s.at[j + 1]], add=True)

            pltpu.make_async_copy(y_hbm.at[src_s.at[SUPER - 1]], rows0,
                                  gsem0).wait()
            pltpu.sync_copy(rows0, acc_sh.at[dst_s.at[SUPER - 1]], add=True)

        plsc.subcore_barrier()
        pltpu.sync_copy(acc_sh.at[pl.ds(r0, ROWS_PER_TILE)],
                        out_hbm.at[cid, pl.ds(r0, ROWS_PER_TILE)])

    return k(y, edges_r)


_RB2 = 5000
_GB = N // _RB2


def _epilogue_body(x_ref, p0_ref, p1_ref, w_ref, sw_ref, b_ref, wt_ref,
                   bt_ref, a_ref, pooled_ref, s_ref, sz_ref):
    i = pl.program_id(0)

    @pl.when(i == 0)
    def _():
        s_ref[...] = jnp.zeros_like(s_ref)
        sz_ref[...] = jnp.zeros_like(sz_ref)

    y = jnp.dot(x_ref[...], w_ref[...], preferred_element_type=jnp.float32)
    agg = jnp.dot(p0_ref[0] + p1_ref[0], w_ref[...],
                  preferred_element_type=jnp.float32)
    h = _selu(y * sw_ref[...] + agg + b_ref[...])
    logits = jnp.dot(h, wt_ref[...],
                     preferred_element_type=jnp.float32) + bt_ref[...]
    m = jnp.max(logits, axis=-1, keepdims=True)
    e = jnp.exp(logits - m)
    a = e / jnp.sum(e, axis=-1, keepdims=True)
    a_ref[...] = a
    s_ref[...] += lax.dot_general(a, h, (((0,), (0,)), ((), ())),
                                  preferred_element_type=jnp.float32)
    ones = jnp.ones((_RB2, D), jnp.float32)
    sz_ref[...] += lax.dot_general(a, ones, (((0,), (0,)), ((), ())),
                                   preferred_element_type=jnp.float32)

    @pl.when(i == _GB - 1)
    def _():
        pooled_ref[...] = _selu(s_ref[...] / sz_ref[...])


def _epilogue(x, partials, W, skip_w, b, Wt, bt):
    a, pooled = pl.pallas_call(
        _epilogue_body,
        grid=(_GB,),
        in_specs=[
            pl.BlockSpec((_RB2, D), lambda i: (i, 0)),
            pl.BlockSpec((1, _RB2, D), lambda i: (0, i, 0)),
            pl.BlockSpec((1, _RB2, D), lambda i: (1, i, 0)),
            pl.BlockSpec((D, D), lambda i: (0, 0)),
            pl.BlockSpec((1, D), lambda i: (0, 0)),
            pl.BlockSpec((1, D), lambda i: (0, 0)),
            pl.BlockSpec((D, K), lambda i: (0, 0)),
            pl.BlockSpec((1, K), lambda i: (0, 0)),
        ],
        out_specs=[
            pl.BlockSpec((_RB2, K), lambda i: (i, 0)),
            pl.BlockSpec((K, D), lambda i: (0, 0)),
        ],
        out_shape=[
            jax.ShapeDtypeStruct((N, K), jnp.float32),
            jax.ShapeDtypeStruct((K, D), jnp.float32),
        ],
        scratch_shapes=[
            pltpu.VMEM((K, D), jnp.float32),
            pltpu.VMEM((K, D), jnp.float32),
        ],
    )(x, partials, partials, W, skip_w.reshape(1, D), b.reshape(1, D), Wt,
      bt.reshape(1, K))
    return pooled, a


def kernel(features, edges, adjacency, W, b, skip_w, Wt, bt):
    edges_r = edges.reshape(2, NW, NSUPER, SUPER, CHUNK)
    partials = _sc_segment_sum(features, edges_r)
    pooled, a = _epilogue(features, partials, W, skip_w, b, Wt, bt)
    return pooled, a


# speedup vs baseline: 1.1312x; 1.0023x over previous
import functools

import jax
import jax.numpy as jnp
from jax import lax
from jax.experimental import pallas as pl
from jax.experimental.pallas import tpu as pltpu
from jax.experimental.pallas import tpu_sc as plsc

N = 10000
E = 320000
D = 128
K = 16

NC = 2
NS = 16
NW = NC * NS
CHUNK = 80
EDGES_PER_TILE = E // NW
NCHUNKS = EDGES_PER_TILE // CHUNK
SUPER = 25
NSUPER = NCHUNKS // SUPER
NPAD = 10240
ROWS_PER_TILE = NPAD // NS

_SELU_SCALE = 1.0507009873554805
_SELU_ALPHA = 1.6732632423543772


def _selu(x):
    safe = jnp.minimum(x, 0.0)
    return _SELU_SCALE * jnp.where(x > 0, x, _SELU_ALPHA * (jnp.exp(safe) - 1.0))


def _sc_segment_sum(y, edges_r):
    mesh = plsc.VectorSubcoreMesh(core_axis_name="c", subcore_axis_name="s")

    @functools.partial(
        pl.kernel,
        mesh=mesh,
        out_type=jax.ShapeDtypeStruct((NC, NPAD, D), jnp.float32),
        scratch_types=[
            pltpu.VMEM((SUPER, CHUNK), jnp.int32),
            pltpu.VMEM((SUPER, CHUNK), jnp.int32),
            pltpu.VMEM((SUPER, CHUNK), jnp.int32),
            pltpu.VMEM((SUPER, CHUNK), jnp.int32),
            pltpu.VMEM((CHUNK, D), jnp.float32),
            pltpu.VMEM((CHUNK, D), jnp.float32),
            pltpu.VMEM_SHARED((NPAD, D), jnp.float32),
            pltpu.SemaphoreType.DMA,
            pltpu.SemaphoreType.DMA,
            pltpu.SemaphoreType.DMA,
            pltpu.SemaphoreType.DMA,
        ],
    )
    def k(y_hbm, e_hbm, out_hbm, src_a, src_b, dst_a, dst_b,
          rows0, rows1, acc_sh, gsem0, gsem1, isem_a, isem_b):
        cid = lax.axis_index("c")
        sid = lax.axis_index("s")
        wid = cid * NS + sid
        r0 = sid * ROWS_PER_TILE
        @pl.loop(0, CHUNK)
        def _(r):
            @pl.loop(0, D // 16)
            def _(c):
                rows0[r, pl.ds(c * 16, 16)] = jnp.zeros((16,), jnp.float32)

        @pl.loop(0, ROWS_PER_TILE // CHUNK)
        def _(i):
            pltpu.sync_copy(rows0, acc_sh.at[pl.ds(r0 + i * CHUNK, CHUNK)])

        def idx_bufs(s):
            return (src_a, dst_a, isem_a) if s % 2 == 0 else \
                   (src_b, dst_b, isem_b)

        s0, d0, i0 = idx_bufs(0)
        pltpu.async_copy(e_hbm.at[0, wid, 0], s0, i0)
        pltpu.async_copy(e_hbm.at[1, wid, 0], d0, i0)
        plsc.subcore_barrier()

        for s in range(NSUPER):
            src_s, dst_s, is_s = idx_bufs(s)
            pltpu.make_async_copy(e_hbm.at[0, wid, s], src_s, is_s).wait()
            pltpu.make_async_copy(e_hbm.at[1, wid, s], dst_s, is_s).wait()
            if s + 1 < NSUPER:
                sn, dn, i_n = idx_bufs(s + 1)
                pltpu.async_copy(e_hbm.at[0, wid, s + 1], sn, i_n)
                pltpu.async_copy(e_hbm.at[1, wid, s + 1], dn, i_n)

            pltpu.async_copy(y_hbm.at[src_s.at[0]], rows0, gsem0)

            @pl.loop(0, SUPER - 1, step=2)
            def _(j):
                pltpu.async_copy(y_hbm.at[src_s.at[j + 1]], rows1, gsem1)
                pltpu.make_async_copy(y_hbm.at[src_s.at[j]], rows0,
                                      gsem0).wait()
                pltpu.sync_copy(rows0, acc_sh.at[dst_s.at[j]], add=True)
                pltpu.async_copy(y_hbm.at[src_s.at[j + 2]], rows0, gsem0)
                pltpu.make_async_copy(y_hbm.at[src_s.at[j + 1]], rows1,
                                      gsem1).wait()
                pltpu.sync_copy(rows1, acc_sh.at[dst_s.at[j + 1]], add=True)

            pltpu.make_async_copy(y_hbm.at[src_s.at[SUPER - 1]], rows0,
                                  gsem0).wait()
            pltpu.sync_copy(rows0, acc_sh.at[dst_s.at[SUPER - 1]], add=True)

        plsc.subcore_barrier()
        pltpu.sync_copy(acc_sh.at[pl.ds(r0, ROWS_PER_TILE)],
                        out_hbm.at[cid, pl.ds(r0, ROWS_PER_TILE)])

    return k(y, edges_r)


_RB2 = 2000
_GB = N // _RB2


def _epilogue_body(x_ref, p0_ref, p1_ref, w_ref, sw_ref, b_ref, wt_ref,
                   bt_ref, a_ref, pooled_ref, s_ref, sz_ref):
    i = pl.program_id(0)

    @pl.when(i == 0)
    def _():
        s_ref[...] = jnp.zeros_like(s_ref)
        sz_ref[...] = jnp.zeros_like(sz_ref)

    y = jnp.dot(x_ref[...], w_ref[...], preferred_element_type=jnp.float32)
    agg = jnp.dot(p0_ref[0] + p1_ref[0], w_ref[...],
                  preferred_element_type=jnp.float32)
    h = _selu(y * sw_ref[...] + agg + b_ref[...])
    logits = jnp.dot(h, wt_ref[...],
                     preferred_element_type=jnp.float32) + bt_ref[...]
    m = jnp.max(logits, axis=-1, keepdims=True)
    e = jnp.exp(logits - m)
    a = e / jnp.sum(e, axis=-1, keepdims=True)
    a_ref[...] = a
    s_ref[...] += lax.dot_general(a, h, (((0,), (0,)), ((), ())),
                                  preferred_element_type=jnp.float32)
    ones = jnp.ones((_RB2, D), jnp.float32)
    sz_ref[...] += lax.dot_general(a, ones, (((0,), (0,)), ((), ())),
                                   preferred_element_type=jnp.float32)

    @pl.when(i == _GB - 1)
    def _():
        pooled_ref[...] = _selu(s_ref[...] / sz_ref[...])


def _epilogue(x, partials, W, skip_w, b, Wt, bt):
    a, pooled = pl.pallas_call(
        _epilogue_body,
        grid=(_GB,),
        in_specs=[
            pl.BlockSpec((_RB2, D), lambda i: (i, 0)),
            pl.BlockSpec((1, _RB2, D), lambda i: (0, i, 0)),
            pl.BlockSpec((1, _RB2, D), lambda i: (1, i, 0)),
            pl.BlockSpec((D, D), lambda i: (0, 0)),
            pl.BlockSpec((1, D), lambda i: (0, 0)),
            pl.BlockSpec((1, D), lambda i: (0, 0)),
            pl.BlockSpec((D, K), lambda i: (0, 0)),
            pl.BlockSpec((1, K), lambda i: (0, 0)),
        ],
        out_specs=[
            pl.BlockSpec((_RB2, K), lambda i: (i, 0)),
            pl.BlockSpec((K, D), lambda i: (0, 0)),
        ],
        out_shape=[
            jax.ShapeDtypeStruct((N, K), jnp.float32),
            jax.ShapeDtypeStruct((K, D), jnp.float32),
        ],
        scratch_shapes=[
            pltpu.VMEM((K, D), jnp.float32),
            pltpu.VMEM((K, D), jnp.float32),
        ],
    )(x, partials, partials, W, skip_w.reshape(1, D), b.reshape(1, D), Wt,
      bt.reshape(1, K))
    return pooled, a


def kernel(features, edges, adjacency, W, b, skip_w, Wt, bt):
    edges_r = edges.reshape(2, NW, NSUPER, SUPER, CHUNK)
    partials = _sc_segment_sum(features, edges_r)
    pooled, a = _epilogue(features, partials, W, skip_w, b, Wt, bt)
    return pooled, a
